# Initial kernel scaffold; baseline (speedup 1.0000x reference)
import jax
import jax.numpy as jnp
from jax.experimental import pallas as pl


def kernel(features, memory_bank):
    raise NotImplementedError("write your pallas kernel here")


# speedup vs baseline: 1.3261x; 1.3261x over previous
import jax
import jax.numpy as jnp
from jax.experimental import pallas as pl
from jax.experimental.pallas import tpu as pltpu

Q, D, K = 3136, 1536, 16384
BQ, BK = 224, 512
KBLOCKS = K // BK
TOPK = 3


def _body(x_ref, yt_ref, val_ref, idx_ref):
    kj = pl.program_id(1)

    x = x_ref[...]
    yt = yt_ref[...]
    d = jax.lax.dot_general(
        x, yt, (((1,), (0,)), ((), ())),
        preferred_element_type=jnp.float32,
    )
    y2 = jnp.sum(yt * yt, axis=0, keepdims=True)
    s = y2 - 2.0 * d

    @pl.when(kj == 0)
    def _init():
        val_ref[...] = jnp.full((BQ, TOPK), jnp.inf, jnp.float32)
        idx_ref[...] = jnp.zeros((BQ, TOPK), jnp.int32)

    iota = jax.lax.broadcasted_iota(jnp.int32, (BQ, BK), 1)
    base = kj * BK
    bvals, bidxs = [], []
    scur = s
    for t in range(TOPK):
        m = jnp.min(scur, axis=1, keepdims=True)
        loc = jnp.min(jnp.where(scur == m, iota, K), axis=1, keepdims=True)
        bvals.append(m)
        bidxs.append(loc + base)
        if t < TOPK - 1:
            scur = jnp.where(iota == loc, jnp.inf, scur)

    v6 = jnp.concatenate([val_ref[...]] + bvals, axis=1)
    i6 = jnp.concatenate([idx_ref[...]] + bidxs, axis=1)
    iota6 = jax.lax.broadcasted_iota(jnp.int32, (BQ, 2 * TOPK), 1)
    outv, outi = [], []
    for t in range(TOPK):
        m = jnp.min(v6, axis=1, keepdims=True)
        pos = jnp.min(jnp.where(v6 == m, iota6, 2 * TOPK), axis=1, keepdims=True)
        sel = iota6 == pos
        outv.append(m)
        outi.append(jnp.sum(jnp.where(sel, i6, 0), axis=1, keepdims=True))
        if t < TOPK - 1:
            v6 = jnp.where(sel, jnp.inf, v6)
    val_ref[...] = jnp.concatenate(outv, axis=1)
    idx_ref[...] = jnp.concatenate(outi, axis=1)

    @pl.when(kj == KBLOCKS - 1)
    def _finish():
        x2 = jnp.sum(x * x, axis=1, keepdims=True)
        val_ref[...] = jnp.sqrt(jnp.maximum(x2 + val_ref[...], 1e-12))


def kernel(features, memory_bank):
    mb_t = memory_bank.T
    vals, idxs = pl.pallas_call(
        _body,
        grid=(Q // BQ, KBLOCKS),
        in_specs=[
            pl.BlockSpec((BQ, D), lambda qi, kj: (qi, 0)),
            pl.BlockSpec((D, BK), lambda qi, kj: (0, kj)),
        ],
        out_specs=[
            pl.BlockSpec((BQ, TOPK), lambda qi, kj: (qi, 0)),
            pl.BlockSpec((BQ, TOPK), lambda qi, kj: (qi, 0)),
        ],
        out_shape=[
            jax.ShapeDtypeStruct((Q, TOPK), jnp.float32),
            jax.ShapeDtypeStruct((Q, TOPK), jnp.int32),
        ],
        compiler_params=pltpu.CompilerParams(
            dimension_semantics=("parallel", "arbitrary"),
        ),
    )(features, mb_t)
    return vals, idxs


# speedup vs baseline: 1.8243x; 1.3757x over previous
import jax
import jax.numpy as jnp
from jax.experimental import pallas as pl
from jax.experimental.pallas import tpu as pltpu

Q, D, K = 3136, 1536, 16384
BQ, BK = 256, 512
QP = 3328
QBLOCKS = QP // BQ
KBLOCKS = K // BK
TOPK = 3


def _body(xt_ref, y_ref, ones_ref, val_ref, idx_ref, sv_ref, si_ref):
    kj = pl.program_id(1)

    xt = xt_ref[...]
    y = y_ref[...]
    d = jax.lax.dot_general(
        y, xt, (((1,), (0,)), ((), ())),
        preferred_element_type=jnp.float32,
    )
    y2 = jax.lax.dot_general(
        y * y, ones_ref[...], (((1,), (0,)), ((), ())),
        preferred_element_type=jnp.float32,
    )[:, 0:1]
    s = y2 - 2.0 * d

    @pl.when(kj == 0)
    def _init():
        sv_ref[...] = jnp.full((8, BQ), jnp.inf, jnp.float32)
        si_ref[...] = jnp.zeros((8, BQ), jnp.int32)

    iota = jax.lax.broadcasted_iota(jnp.int32, (BK, BQ), 0)
    base = kj * BK
    bvals, bidxs = [], []
    scur = s
    for t in range(TOPK):
        m = jnp.min(scur, axis=0, keepdims=True)
        loc = jnp.min(jnp.where(scur == m, iota, BK), axis=0,
                      keepdims=True)
        bvals.append(m)
        bidxs.append(loc + base)
        if t < TOPK - 1:
            scur = jnp.where(iota == loc, jnp.inf, scur)

    v6 = jnp.concatenate([sv_ref[0:TOPK]] + bvals, axis=0)
    i6 = jnp.concatenate([si_ref[0:TOPK]] + bidxs, axis=0)
    iota6 = jax.lax.broadcasted_iota(jnp.int32, (2 * TOPK, BQ), 0)
    outv, outi = [], []
    for t in range(TOPK):
        m = jnp.min(v6, axis=0, keepdims=True)
        pos = jnp.min(jnp.where(v6 == m, iota6, 2 * TOPK), axis=0,
                      keepdims=True)
        sel = iota6 == pos
        outv.append(m)
        outi.append(jnp.sum(jnp.where(sel, i6, 0), axis=0, keepdims=True))
        if t < TOPK - 1:
            v6 = jnp.where(sel, jnp.inf, v6)
    sv_ref[0:TOPK] = jnp.concatenate(outv, axis=0)
    si_ref[0:TOPK] = jnp.concatenate(outi, axis=0)

    @pl.when(kj == KBLOCKS - 1)
    def _finish():
        x2 = jnp.sum(xt * xt, axis=0, keepdims=True)
        val_ref[...] = jnp.sqrt(jnp.maximum(sv_ref[0:TOPK] + x2, 1e-12))
        idx_ref[...] = si_ref[0:TOPK]


def kernel(features, memory_bank):
    xt = jnp.pad(features.T, ((0, 0), (0, QP - Q)))
    ones = jnp.ones((D, 8), jnp.float32)
    vals_t, idxs_t = pl.pallas_call(
        _body,
        grid=(QBLOCKS, KBLOCKS),
        in_specs=[
            pl.BlockSpec((D, BQ), lambda qi, kj: (0, qi)),
            pl.BlockSpec((BK, D), lambda qi, kj: (kj, 0)),
            pl.BlockSpec((D, 8), lambda qi, kj: (0, 0)),
        ],
        out_specs=[
            pl.BlockSpec((TOPK, BQ), lambda qi, kj: (0, qi)),
            pl.BlockSpec((TOPK, BQ), lambda qi, kj: (0, qi)),
        ],
        out_shape=[
            jax.ShapeDtypeStruct((TOPK, QP), jnp.float32),
            jax.ShapeDtypeStruct((TOPK, QP), jnp.int32),
        ],
        scratch_shapes=[
            pltpu.VMEM((8, BQ), jnp.float32),
            pltpu.VMEM((8, BQ), jnp.int32),
        ],
        compiler_params=pltpu.CompilerParams(
            dimension_semantics=("parallel", "arbitrary"),
        ),
    )(xt, memory_bank, ones)
    return vals_t[:, :Q].T, idxs_t[:, :Q].T


# speedup vs baseline: 1.8763x; 1.0285x over previous
import jax
import jax.numpy as jnp
from jax.experimental import pallas as pl
from jax.experimental.pallas import tpu as pltpu

Q, D, K = 3136, 1536, 16384
BQ, BK = 256, 512
QP = 3328
QBLOCKS = QP // BQ
KBLOCKS = K // BK
TOPK = 3


def _body(x_ref, yt_ref, val_ref, idx_ref, sv_ref, si_ref):
    kj = pl.program_id(1)

    x = x_ref[...]
    yt = yt_ref[...]
    d = jax.lax.dot_general(
        x, yt, (((1,), (0,)), ((), ())),
        preferred_element_type=jnp.float32,
    )
    y2 = jnp.sum(yt * yt, axis=0, keepdims=True)
    s = y2 - 2.0 * d
    st = s.T

    @pl.when(kj == 0)
    def _init():
        sv_ref[...] = jnp.full((8, BQ), jnp.inf, jnp.float32)
        si_ref[...] = jnp.zeros((8, BQ), jnp.int32)

    iota = jax.lax.broadcasted_iota(jnp.int32, (BK, BQ), 0)
    base = kj * BK
    bvals, bidxs = [], []
    scur = st
    for t in range(TOPK):
        m = jnp.min(scur, axis=0, keepdims=True)
        loc = jnp.min(jnp.where(scur == m, iota, BK), axis=0,
                      keepdims=True)
        bvals.append(m)
        bidxs.append(loc + base)
        if t < TOPK - 1:
            scur = jnp.where(iota == loc, jnp.inf, scur)

    v6 = jnp.concatenate([sv_ref[0:TOPK]] + bvals, axis=0)
    i6 = jnp.concatenate([si_ref[0:TOPK]] + bidxs, axis=0)
    iota6 = jax.lax.broadcasted_iota(jnp.int32, (2 * TOPK, BQ), 0)
    outv, outi = [], []
    for t in range(TOPK):
        m = jnp.min(v6, axis=0, keepdims=True)
        pos = jnp.min(jnp.where(v6 == m, iota6, 2 * TOPK), axis=0,
                      keepdims=True)
        sel = iota6 == pos
        outv.append(m)
        outi.append(jnp.sum(jnp.where(sel, i6, 0), axis=0, keepdims=True))
        if t < TOPK - 1:
            v6 = jnp.where(sel, jnp.inf, v6)
    sv_ref[0:TOPK] = jnp.concatenate(outv, axis=0)
    si_ref[0:TOPK] = jnp.concatenate(outi, axis=0)

    @pl.when(kj == KBLOCKS - 1)
    def _finish():
        x2 = jnp.sum(x * x, axis=1, keepdims=True).T
        val_ref[...] = jnp.sqrt(jnp.maximum(sv_ref[0:TOPK] + x2, 1e-12))
        idx_ref[...] = si_ref[0:TOPK]


def kernel(features, memory_bank):
    xp = jnp.pad(features, ((0, QP - Q), (0, 0)))
    mb_t = memory_bank.T
    vals_t, idxs_t = pl.pallas_call(
        _body,
        grid=(QBLOCKS, KBLOCKS),
        in_specs=[
            pl.BlockSpec((BQ, D), lambda qi, kj: (qi, 0)),
            pl.BlockSpec((D, BK), lambda qi, kj: (0, kj)),
        ],
        out_specs=[
            pl.BlockSpec((TOPK, BQ), lambda qi, kj: (0, qi)),
            pl.BlockSpec((TOPK, BQ), lambda qi, kj: (0, qi)),
        ],
        out_shape=[
            jax.ShapeDtypeStruct((TOPK, QP), jnp.float32),
            jax.ShapeDtypeStruct((TOPK, QP), jnp.int32),
        ],
        scratch_shapes=[
            pltpu.VMEM((8, BQ), jnp.float32),
            pltpu.VMEM((8, BQ), jnp.int32),
        ],
        compiler_params=pltpu.CompilerParams(
            dimension_semantics=("parallel", "arbitrary"),
        ),
    )(xp, mb_t)
    return vals_t[:, :Q].T, idxs_t[:, :Q].T


# speedup vs baseline: 2.7651x; 1.4737x over previous
import jax
import jax.numpy as jnp
from jax.experimental import pallas as pl
from jax.experimental.pallas import tpu as pltpu

Q, D, K = 3136, 1536, 16384
BQ, BK = 256, 2048
QP = 3328
QBLOCKS = QP // BQ
KBLOCKS = K // BK
TOPK = 3


def _body(x_ref, yt_ref, val_ref, idx_ref, sv_ref, si_ref, y2_ref):
    kj = pl.program_id(0)
    qi = pl.program_id(1)
    qs = pl.ds(qi * BQ, BQ)

    @pl.when(qi == 0)
    def _cache_y2():
        yt = yt_ref[...]
        y2_ref[0:1, :] = jnp.sum(yt * yt, axis=0, keepdims=True)

    @pl.when(kj == 0)
    def _init():
        sv_ref[:, qs] = jnp.full((8, BQ), jnp.inf, jnp.float32)
        si_ref[:, qs] = jnp.zeros((8, BQ), jnp.int32)

    x = x_ref[...]
    d = jax.lax.dot_general(
        x, yt_ref[...], (((1,), (0,)), ((), ())),
        preferred_element_type=jnp.float32,
    )
    s = y2_ref[0:1, :] - 2.0 * d
    st = s.T

    iota = jax.lax.broadcasted_iota(jnp.int32, (BK, BQ), 0)
    base = kj * BK
    bvals, bidxs = [], []
    scur = st
    for t in range(TOPK):
        m = jnp.min(scur, axis=0, keepdims=True)
        loc = jnp.min(jnp.where(scur == m, iota, BK), axis=0,
                      keepdims=True)
        bvals.append(m)
        bidxs.append(loc + base)
        if t < TOPK - 1:
            scur = jnp.where(iota == loc, jnp.inf, scur)

    v6 = jnp.concatenate([sv_ref[0:TOPK, qs]] + bvals, axis=0)
    i6 = jnp.concatenate([si_ref[0:TOPK, qs]] + bidxs, axis=0)
    iota6 = jax.lax.broadcasted_iota(jnp.int32, (2 * TOPK, BQ), 0)
    outv, outi = [], []
    for t in range(TOPK):
        m = jnp.min(v6, axis=0, keepdims=True)
        pos = jnp.min(jnp.where(v6 == m, iota6, 2 * TOPK), axis=0,
                      keepdims=True)
        sel = iota6 == pos
        outv.append(m)
        outi.append(jnp.sum(jnp.where(sel, i6, 0), axis=0, keepdims=True))
        if t < TOPK - 1:
            v6 = jnp.where(sel, jnp.inf, v6)
    sv_ref[0:TOPK, qs] = jnp.concatenate(outv, axis=0)
    si_ref[0:TOPK, qs] = jnp.concatenate(outi, axis=0)

    @pl.when(kj == KBLOCKS - 1)
    def _finish():
        x2 = jnp.sum(x * x, axis=1, keepdims=True).T
        val_ref[...] = jnp.sqrt(
            jnp.maximum(sv_ref[0:TOPK, qs] + x2, 1e-12))
        idx_ref[...] = si_ref[0:TOPK, qs]


def kernel(features, memory_bank):
    xp = jnp.pad(features, ((0, QP - Q), (0, 0)))
    mb_t = memory_bank.T
    vals_t, idxs_t = pl.pallas_call(
        _body,
        grid=(KBLOCKS, QBLOCKS),
        in_specs=[
            pl.BlockSpec((BQ, D), lambda kj, qi: (qi, 0)),
            pl.BlockSpec((D, BK), lambda kj, qi: (0, kj)),
        ],
        out_specs=[
            pl.BlockSpec((TOPK, BQ), lambda kj, qi: (0, qi)),
            pl.BlockSpec((TOPK, BQ), lambda kj, qi: (0, qi)),
        ],
        out_shape=[
            jax.ShapeDtypeStruct((TOPK, QP), jnp.float32),
            jax.ShapeDtypeStruct((TOPK, QP), jnp.int32),
        ],
        scratch_shapes=[
            pltpu.VMEM((8, QP), jnp.float32),
            pltpu.VMEM((8, QP), jnp.int32),
            pltpu.VMEM((8, BK), jnp.float32),
        ],
        compiler_params=pltpu.CompilerParams(
            dimension_semantics=("arbitrary", "arbitrary"),
        ),
    )(xp, mb_t)
    return vals_t[:, :Q].T, idxs_t[:, :Q].T
